# Initial kernel scaffold; baseline (speedup 1.0000x reference)
import jax
import jax.numpy as jnp
from jax.experimental import pallas as pl


def kernel(x, edge_index, edge_attr, params):
    raise NotImplementedError("write your pallas kernel here")


# speedup vs baseline: 11.4705x; 11.4705x over previous
import functools

import jax
import jax.numpy as jnp
from jax import lax
from jax.experimental import pallas as pl
from jax.experimental.pallas import tpu as pltpu, tpu_sc as plsc

B, N, D_IN, E, ED, H = 2, 50176, 7, 401408, 5, 64
K, HEADS, C, HT, WD = 16, 4, 7, 224, 224

RT = 1024
NT = N // RT
ET = 2048

N4 = N // 4
CH = 64
NSUB = 16
EPT = E // NSUB
RPT = N4 // NSUB
NPT = N // 32


def _ln(x, g, b):
    m = jnp.mean(x, axis=-1, keepdims=True)
    v = jnp.mean((x - m) ** 2, axis=-1, keepdims=True)
    return (x - m) * jax.lax.rsqrt(v + 1e-5) * g + b


def _gelu(x):
    return x * 0.5 * (1.0 + lax.erf(x * 0.7071067811865476))


def _dot_t(x, w):
    return lax.dot_general(x, w, (((1,), (1,)), ((), ())),
                           preferred_element_type=jnp.float32)


def _stage1_body(xa_ref, xb_ref, in_w, in_b, in_g, in_lb, msg_w, msg_b,
                 ha_ref, hb_ref, hm_ref):
    ha = _gelu(_ln(_dot_t(xa_ref[...], in_w[...]) + in_b[...],
                   in_g[...], in_lb[...]))
    hb = _gelu(_ln(_dot_t(xb_ref[...], in_w[...]) + in_b[...],
                   in_g[...], in_lb[...]))
    ha_ref[...] = ha
    hb_ref[...] = hb
    hm_ref[...] = jnp.concatenate(
        [_gelu(_dot_t(ha, msg_w[...]) + msg_b[...]),
         _gelu(_dot_t(hb, msg_w[...]) + msg_b[...])], axis=1)


def _stage1(x2, p):
    full = lambda s: pl.BlockSpec(s, lambda i: (0,) * len(s))
    return pl.pallas_call(
        _stage1_body,
        grid=(NT,),
        in_specs=[
            pl.BlockSpec((RT, D_IN), lambda i: (i, 0)),
            pl.BlockSpec((RT, D_IN), lambda i: (NT + i, 0)),
            full((H, D_IN)), full((1, H)), full((1, H)), full((1, H)),
            full((H, H)), full((1, H)),
        ],
        out_specs=[
            pl.BlockSpec((RT, H), lambda i: (i, 0)),
            pl.BlockSpec((RT, H), lambda i: (i, 0)),
            pl.BlockSpec((RT, 2 * H), lambda i: (i, 0)),
        ],
        out_shape=[
            jax.ShapeDtypeStruct((N, H), jnp.float32),
            jax.ShapeDtypeStruct((N, H), jnp.float32),
            jax.ShapeDtypeStruct((N, 2 * H), jnp.float32),
        ],
    )(x2, x2, p['in_w'], p['in_b'].reshape(1, H), p['in_g'].reshape(1, H),
      p['in_lb'].reshape(1, H), p['msg_w'], p['msg_b'].reshape(1, H))


def _stage2_body(ea_ref, w1, b1, w2, b2, gate_ref):
    t = _gelu(_dot_t(ea_ref[...], w1[...]) + b1[...])
    gate_ref[...] = jax.nn.sigmoid(_dot_t(t, w2[...]) + b2[...])


def _stage2(edge_attr, p):
    full = lambda s: pl.BlockSpec(s, lambda i: (0,) * len(s))
    return pl.pallas_call(
        _stage2_body,
        grid=(E // ET,),
        in_specs=[
            pl.BlockSpec((ET, ED), lambda i: (i, 0)),
            full((H, ED)), full((1, H)), full((H, H)), full((1, H)),
        ],
        out_specs=pl.BlockSpec((ET, H), lambda i: (i, 0)),
        out_shape=jax.ShapeDtypeStruct((E, H), jnp.float32),
    )(edge_attr, p['eg1_w'], p['eg1_b'].reshape(1, H), p['eg2_w'],
      p['eg2_b'].reshape(1, H))


DR = N // 128


def _edge_body(hm_hbm, gate_hbm, src_hbm, dst_hbm, id_hbm, agg_hbm, deg_hbm,
               srcv, dstv, idxa, gatev, rowsv, acc, deg2, sem):
    c = lax.axis_index("c")
    s = lax.axis_index("s")

    for p in range(2):
        lo = (c * 2 + p) * N4
        plsc.subcore_barrier()

        def _z(i, _):
            rowsv[i // 8, pl.ds((i % 8) * 16, 16)] = jnp.zeros((16,),
                                                              jnp.float32)
            return 0
        lax.fori_loop(0, CH * 8, _z, 0)
        for z in range(RPT // CH):
            pltpu.sync_copy(rowsv, acc.at[pl.ds(s * RPT + z * CH, CH)])
        pltpu.sync_copy(rowsv.at[pl.ds(0, 16)],
                        acc.at[pl.ds(s * RPT + (RPT // CH) * CH, 16)])

        @pl.when(s == 0)
        def _():
            pltpu.sync_copy(rowsv.at[pl.ds(0, 8)], acc.at[pl.ds(N4, 8)])
        if p == 0:
            pltpu.sync_copy(rowsv.at[pl.ds(0, 24)],
                            deg2.at[pl.ds(s * 24, 24)])

            @pl.when(s == 0)
            def _():
                pltpu.sync_copy(rowsv.at[pl.ds(0, 8)],
                                deg2.at[pl.ds(384, 8)])
        plsc.subcore_barrier()

        def chunk(ci, _):
            base = s * EPT + ci * CH
            pltpu.sync_copy(src_hbm.at[pl.ds(base, CH)], srcv)
            pltpu.sync_copy(dst_hbm.at[pl.ds(base, CH)], dstv)
            pltpu.sync_copy(gate_hbm.at[pl.ds(base, CH)], gatev)

            def _i1(j, _):
                d = dstv[pl.ds(j * 16, 16)]
                lcl = d - lo
                m = (lcl >= 0) & (lcl < N4)
                idxa[0, pl.ds(j * 16, 16)] = jnp.where(m, lcl, N4)
                return 0
            lax.fori_loop(0, CH // 16, _i1, 0)
            if p == 0:
                def _i2(j, _):
                    d = dstv[pl.ds(j * 16, 16)]
                    idxa[1, pl.ds(j * 16, 16)] = d >> 7
                    return 0
                lax.fori_loop(0, CH // 16, _i2, 0)

                def _i3(j, _):
                    d = dstv[pl.ds(j * 16, 16)]
                    idxa[2, pl.ds(j * 16, 16)] = d & 127
                    return 0
                lax.fori_loop(0, CH // 16, _i3, 0)
                pltpu.async_copy(id_hbm.at[idxa.at[2]], rowsv, sem).wait()
                pltpu.sync_copy(rowsv, deg2.at[idxa.at[1]], add=True)
            pltpu.async_copy(hm_hbm.at[srcv], rowsv, sem).wait()

            def _mul(e, _):
                g = [gatev[e, pl.ds(kk * 16, 16)] for kk in range(4)]
                for kk in range(8):
                    rowsv[e, pl.ds(kk * 16, 16)] = (
                        rowsv[e, pl.ds(kk * 16, 16)] * g[kk % 4])
                return 0
            lax.fori_loop(0, CH, _mul, 0)
            pltpu.sync_copy(rowsv, acc.at[idxa.at[0]], add=True)
            return 0
        lax.fori_loop(0, EPT // CH, chunk, 0)
        plsc.subcore_barrier()
        pltpu.sync_copy(acc.at[pl.ds(s * RPT, RPT)],
                        agg_hbm.at[pl.ds(lo + s * RPT, RPT)])
        if p == 0:
            pltpu.sync_copy(deg2.at[pl.ds(s * 24, 24)],
                            deg_hbm.at[c, pl.ds(s * 24, 24)])

            @pl.when(s == 0)
            def _():
                pltpu.sync_copy(deg2.at[pl.ds(384, 8)],
                                deg_hbm.at[c, pl.ds(384, 8)])
    plsc.subcore_barrier()


def _stage3(hm, gate, src, dst, ident):
    mesh = plsc.VectorSubcoreMesh(core_axis_name="c", subcore_axis_name="s")
    f = functools.partial(
        pl.kernel, _edge_body, mesh=mesh,
        out_type=[jax.ShapeDtypeStruct((N, 2 * H), jnp.float32),
                  jax.ShapeDtypeStruct((2, DR, 128), jnp.float32)],
        scratch_types=[
            pltpu.VMEM((CH,), jnp.int32),
            pltpu.VMEM((CH,), jnp.int32),
            pltpu.VMEM((3, CH), jnp.int32),
            pltpu.VMEM((CH, H), jnp.float32),
            pltpu.VMEM((CH, 2 * H), jnp.float32),
            pltpu.VMEM_SHARED((N4 + 8, 2 * H), jnp.float32),
            pltpu.VMEM_SHARED((DR + 8, 128), jnp.float32),
            pltpu.SemaphoreType.DMA,
        ],
    )()
    return f(hm, gate, src, dst, ident)


def _stage4_body(ha_ref, hb_ref, agg_ref, deg_ref, agg_w, agg_bb, nm_g,
                 nm_b, f1_w, f1_b, f2_w, f2_b, nf_g, nf_b, key_w, key_b,
                 pq, val_w, val_b, s_ref):
    i = pl.program_id(0)
    a = agg_ref[...]
    nloc = lax.broadcasted_iota(jnp.int32, (RT, 1), 0)
    rsel = (nloc // 128 == lax.broadcasted_iota(jnp.int32, (RT, 8), 1)
            ).astype(jnp.float32)
    d8 = 0.5 * jnp.sum(deg_ref[...], axis=0)
    dflat = jnp.dot(rsel, d8, preferred_element_type=jnp.float32)
    lsel = (nloc % 128 == lax.broadcasted_iota(jnp.int32, (RT, 128), 1))
    deg = jnp.sum(jnp.where(lsel, dflat, 0.0), axis=1, keepdims=True)
    rdeg = 1.0 / jnp.clip(deg, 1.0, None)
    n = i * RT + nloc
    y = (n // WD).astype(jnp.float32) * (1.0 / (HT - 1))
    x = (n % WD).astype(jnp.float32) * (1.0 / (WD - 1))
    lane = lax.broadcasted_iota(jnp.int32, (RT, H), 1)
    aux = (jnp.where(lane == 0, y, 0.0) + jnp.where(lane == 1, x, 0.0)
           + jnp.where(lane == 2, 1.0, 0.0))

    @pl.when(i == 0)
    def _():
        s_ref[...] = jnp.zeros_like(s_ref)

    for b, h_ref in ((0, ha_ref), (1, hb_ref)):
        aggn = a[:, b * H:(b + 1) * H] * rdeg
        t = _gelu(_dot_t(aggn, agg_w[...]) + agg_bb[...])
        h2 = _ln(h_ref[...] + t, nm_g[...], nm_b[...])
        ffn = (_dot_t(_gelu(_dot_t(h2, f1_w[...]) + f1_b[...]), f2_w[...])
               + f2_b[...])
        h3 = _ln(h2 + ffn, nf_g[...], nf_b[...])
        keys = _dot_t(h3, key_w[...]) + key_b[...]
        logits = _dot_t(keys, pq[...]) * 0.125
        z = logits - jnp.max(logits, axis=-1, keepdims=True)
        ez = jnp.exp(z)
        m = ez / jnp.sum(ez, axis=-1, keepdims=True)
        vals = _dot_t(h3, val_w[...]) + val_b[...]
        va = jnp.concatenate([vals, aux], axis=1)
        part = lax.dot_general(m, va, (((0,), (0,)), ((), ())),
                               preferred_element_type=jnp.float32)
        s_ref[b, :, :] += part


def _stage4(ha, hb, agg, deg, p):
    full = lambda s: pl.BlockSpec(s, lambda i: (0,) * len(s))
    r1 = lambda name: p[name].reshape(1, -1)
    return pl.pallas_call(
        _stage4_body,
        grid=(NT,),
        in_specs=[
            pl.BlockSpec((RT, H), lambda i: (i, 0)),
            pl.BlockSpec((RT, H), lambda i: (i, 0)),
            pl.BlockSpec((RT, 2 * H), lambda i: (i, 0)),
            pl.BlockSpec((2, 8, 128), lambda i: (0, i, 0)),
            full((H, H)), full((1, H)), full((1, H)), full((1, H)),
            full((2 * H, H)), full((1, 2 * H)), full((H, 2 * H)),
            full((1, H)), full((1, H)), full((1, H)),
            full((H, H)), full((1, H)), full((K, H)),
            full((H, H)), full((1, H)),
        ],
        out_specs=pl.BlockSpec((B, K, 2 * H), lambda i: (0, 0, 0)),
        out_shape=jax.ShapeDtypeStruct((B, K, 2 * H), jnp.float32),
    )(ha, hb, agg, deg, p['agg_w'], r1('agg_b'), r1('nm_g'), r1('nm_b'),
      p['f1_w'], r1('f1_b'), p['f2_w'], r1('f2_b'), r1('nf_g'), r1('nf_b'),
      p['key_w'], r1('key_b'), p['pq'], p['val_w'], r1('val_b'))


def _stage5_body(s_ref, pos1_wt, pos1_b, pos2_w, pos2_b,
                 qw, qb, kw, kb, vw, vb, ao_w, ao_b, na_g, na_b,
                 pf1_w, pf1_b, pf2_w, pf2_b, np_g, np_b,
                 c1_w, c1_b, c2_w, c2_b, out_ref):
    lane = lax.broadcasted_iota(jnp.int32, (K, 2 * H), 1)
    for b in range(B):
        sb = s_ref[b]
        den = jnp.sum(jnp.where(lane == H + 2, sb, 0.0), axis=1,
                      keepdims=True) + 1e-8
        cy = jnp.sum(jnp.where(lane == H, sb, 0.0), axis=1, keepdims=True)
        cx = jnp.sum(jnp.where(lane == H + 1, sb, 0.0), axis=1, keepdims=True)
        pf = sb[:, 0:H] / den
        pe = _gelu((cy / den) * pos1_wt[0:1, :] + (cx / den) * pos1_wt[1:2, :]
                   + pos1_b[...])
        pf = pf + _dot_t(pe, pos2_w[...]) + pos2_b[...]
        q = _dot_t(pf, qw[...]) + qb[...]
        k = _dot_t(pf, kw[...]) + kb[...]
        v = _dot_t(pf, vw[...]) + vb[...]
        dh = H // HEADS
        aos = []
        for hh in range(HEADS):
            qh = q[:, hh * dh:(hh + 1) * dh]
            kh = k[:, hh * dh:(hh + 1) * dh]
            vh = v[:, hh * dh:(hh + 1) * dh]
            aw = lax.dot_general(qh, kh, (((1,), (1,)), ((), ())),
                                 preferred_element_type=jnp.float32) * 0.25
            aw = aw - jnp.max(aw, axis=-1, keepdims=True)
            ea = jnp.exp(aw)
            aw = ea / jnp.sum(ea, axis=-1, keepdims=True)
            aos.append(jnp.dot(aw, vh, preferred_element_type=jnp.float32))
        ao = jnp.concatenate(aos, axis=1)
        ao = _dot_t(ao, ao_w[...]) + ao_b[...]
        hp = _ln(pf + ao, na_g[...], na_b[...])
        pffn = _dot_t(_gelu(_dot_t(hp, pf1_w[...]) + pf1_b[...]),
                      pf2_w[...]) + pf2_b[...]
        hp = _ln(hp + pffn, np_g[...], np_b[...])
        img = jnp.mean(hp, axis=0, keepdims=True)
        ob = _dot_t(_gelu(_dot_t(img, c1_w[...]) + c1_b[...]),
                    c2_w[...]) + c2_b[...]
        out_ref[b:b + 1, :] = ob


def _stage5(s, p):
    c2p = jnp.zeros((2 * H, 2 * H), jnp.float32).at[:C, :].set(p['c2_w'])
    c2bp = jnp.zeros((1, 2 * H), jnp.float32).at[0, :C].set(p['c2_b'])
    r1 = lambda a: a.reshape(1, -1)
    args = [s, p['pos1_w'].T, r1(p['pos1_b']), p['pos2_w'], r1(p['pos2_b']),
            p['qkv_w'][0:H], r1(p['qkv_b'][0:H]),
            p['qkv_w'][H:2 * H], r1(p['qkv_b'][H:2 * H]),
            p['qkv_w'][2 * H:], r1(p['qkv_b'][2 * H:]),
            p['ao_w'], r1(p['ao_b']), r1(p['na_g']), r1(p['na_b']),
            p['pf1_w'], r1(p['pf1_b']), p['pf2_w'], r1(p['pf2_b']),
            r1(p['np_g']), r1(p['np_b']),
            p['c1_w'], r1(p['c1_b']), c2p, c2bp]
    out = pl.pallas_call(
        _stage5_body,
        in_specs=[pl.BlockSpec(a.shape, (lambda nd: lambda: (0,) * nd)(a.ndim))
                  for a in args],
        out_specs=pl.BlockSpec((B, 2 * H), lambda: (0, 0)),
        out_shape=jax.ShapeDtypeStruct((B, 2 * H), jnp.float32),
    )(*args)
    return out[:, :C]


def kernel(x, edge_index, edge_attr, params):
    p = params
    x2 = x.reshape(B * N, D_IN)
    src = edge_index[0].astype(jnp.int32)
    dst = edge_index[1].astype(jnp.int32)
    ha, hb, hm = _stage1(x2, p)
    gate = _stage2(edge_attr, p)
    ident = jnp.eye(128, dtype=jnp.float32)
    agg, deg = _stage3(hm, gate, src, dst, ident)
    s = _stage4(ha, hb, agg, deg, p)
    return _stage5(s, p)


# speedup vs baseline: 12.6509x; 1.1029x over previous
import functools

import jax
import jax.numpy as jnp
from jax import lax
from jax.experimental import pallas as pl
from jax.experimental.pallas import tpu as pltpu, tpu_sc as plsc

B, N, D_IN, E, ED, H = 2, 50176, 7, 401408, 5, 64
K, HEADS, C, HT, WD = 16, 4, 7, 224, 224

RT = 1024
NT = N // RT
ET = 2048

N4 = N // 4
CH = 64
NSUB = 16
EPT = E // NSUB
RPT = N4 // NSUB
NPT = N // 32


def _ln(x, g, b):
    m = jnp.mean(x, axis=-1, keepdims=True)
    v = jnp.mean((x - m) ** 2, axis=-1, keepdims=True)
    return (x - m) * jax.lax.rsqrt(v + 1e-5) * g + b


def _gelu(x):
    return x * 0.5 * (1.0 + lax.erf(x * 0.7071067811865476))


def _dot_t(x, w):
    return lax.dot_general(x, w, (((1,), (1,)), ((), ())),
                           preferred_element_type=jnp.float32)


def _stage1_body(xa_ref, xb_ref, in_w, in_b, in_g, in_lb, msg_w, msg_b,
                 ha_ref, hb_ref, hm_ref):
    ha = _gelu(_ln(_dot_t(xa_ref[...], in_w[...]) + in_b[...],
                   in_g[...], in_lb[...]))
    hb = _gelu(_ln(_dot_t(xb_ref[...], in_w[...]) + in_b[...],
                   in_g[...], in_lb[...]))
    ha_ref[...] = ha
    hb_ref[...] = hb
    hm_ref[...] = jnp.concatenate(
        [_gelu(_dot_t(ha, msg_w[...]) + msg_b[...]),
         _gelu(_dot_t(hb, msg_w[...]) + msg_b[...])], axis=1)


def _stage1(x2, p):
    full = lambda s: pl.BlockSpec(s, lambda i: (0,) * len(s))
    return pl.pallas_call(
        _stage1_body,
        grid=(NT,),
        in_specs=[
            pl.BlockSpec((RT, D_IN), lambda i: (i, 0)),
            pl.BlockSpec((RT, D_IN), lambda i: (NT + i, 0)),
            full((H, D_IN)), full((1, H)), full((1, H)), full((1, H)),
            full((H, H)), full((1, H)),
        ],
        out_specs=[
            pl.BlockSpec((RT, H), lambda i: (i, 0)),
            pl.BlockSpec((RT, H), lambda i: (i, 0)),
            pl.BlockSpec((RT, 2 * H), lambda i: (i, 0)),
        ],
        out_shape=[
            jax.ShapeDtypeStruct((N, H), jnp.float32),
            jax.ShapeDtypeStruct((N, H), jnp.float32),
            jax.ShapeDtypeStruct((N, 2 * H), jnp.float32),
        ],
    )(x2, x2, p['in_w'], p['in_b'].reshape(1, H), p['in_g'].reshape(1, H),
      p['in_lb'].reshape(1, H), p['msg_w'], p['msg_b'].reshape(1, H))


def _stage2_body(ea_ref, w1, b1, w2, b2, gate_ref):
    t = _gelu(_dot_t(ea_ref[...], w1[...]) + b1[...])
    gate_ref[...] = jax.nn.sigmoid(_dot_t(t, w2[...]) + b2[...])


def _stage2(edge_attr, p):
    full = lambda s: pl.BlockSpec(s, lambda i: (0,) * len(s))
    return pl.pallas_call(
        _stage2_body,
        grid=(E // ET,),
        in_specs=[
            pl.BlockSpec((ET, ED), lambda i: (i, 0)),
            full((H, ED)), full((1, H)), full((H, H)), full((1, H)),
        ],
        out_specs=pl.BlockSpec((ET, H), lambda i: (i, 0)),
        out_shape=jax.ShapeDtypeStruct((E, H), jnp.float32),
    )(edge_attr, p['eg1_w'], p['eg1_b'].reshape(1, H), p['eg2_w'],
      p['eg2_b'].reshape(1, H))


DR = N // 128


def _edge_body(hm_hbm, gate_hbm, src_hbm, dst_hbm, id_hbm, agg_hbm, deg_hbm,
               srcv, dstv, idxa, gatev, rowsv, acc, deg2, sem):
    c = lax.axis_index("c")
    s = lax.axis_index("s")

    for p in range(2):
        lo = (c * 2 + p) * N4
        plsc.subcore_barrier()

        def _z(i, _):
            rowsv[i // 8, pl.ds((i % 8) * 16, 16)] = jnp.zeros((16,),
                                                              jnp.float32)
            return 0
        lax.fori_loop(0, CH * 8, _z, 0)
        for z in range(RPT // CH):
            pltpu.sync_copy(rowsv, acc.at[pl.ds(s * RPT + z * CH, CH)])
        if RPT % CH:
            pltpu.sync_copy(rowsv.at[pl.ds(0, RPT % CH)],
                            acc.at[pl.ds(s * RPT + (RPT // CH) * CH,
                                         RPT % CH)])

        @pl.when(s == 0)
        def _():
            pltpu.sync_copy(rowsv.at[pl.ds(0, 8)], acc.at[pl.ds(N4, 8)])
        if p == 0:
            pltpu.sync_copy(rowsv.at[pl.ds(0, 24)],
                            deg2.at[pl.ds(s * 24, 24)])

            @pl.when(s == 0)
            def _():
                pltpu.sync_copy(rowsv.at[pl.ds(0, 8)],
                                deg2.at[pl.ds(384, 8)])
        plsc.subcore_barrier()

        def chunk(ci, _):
            base = s * EPT + ci * CH
            pltpu.sync_copy(src_hbm.at[pl.ds(base, CH)], srcv)
            hmcp = None
            if p == 1:
                hmcp = pltpu.async_copy(hm_hbm.at[srcv], rowsv, sem)
            pltpu.sync_copy(dst_hbm.at[pl.ds(base, CH)], dstv)
            pltpu.sync_copy(gate_hbm.at[pl.ds(base, CH)], gatev)

            def _i1(j, _):
                d = dstv[pl.ds(j * 16, 16)]
                lcl = d - lo
                m = (lcl >= 0) & (lcl < N4)
                idxa[0, pl.ds(j * 16, 16)] = jnp.where(m, lcl, N4)
                return 0
            lax.fori_loop(0, CH // 16, _i1, 0)
            if p == 0:
                def _i2(j, _):
                    d = dstv[pl.ds(j * 16, 16)]
                    idxa[1, pl.ds(j * 16, 16)] = d >> 7
                    return 0
                lax.fori_loop(0, CH // 16, _i2, 0)

                def _i3(j, _):
                    d = dstv[pl.ds(j * 16, 16)]
                    idxa[2, pl.ds(j * 16, 16)] = d & 127
                    return 0
                lax.fori_loop(0, CH // 16, _i3, 0)
                pltpu.async_copy(id_hbm.at[idxa.at[2]], rowsv, sem).wait()
                pltpu.sync_copy(rowsv, deg2.at[idxa.at[1]], add=True)
            if hmcp is None:
                hmcp = pltpu.async_copy(hm_hbm.at[srcv], rowsv, sem)
            hmcp.wait()

            def _mul(e, _):
                g = [gatev[e, pl.ds(kk * 16, 16)] for kk in range(4)]
                for kk in range(8):
                    rowsv[e, pl.ds(kk * 16, 16)] = (
                        rowsv[e, pl.ds(kk * 16, 16)] * g[kk % 4])
                return 0
            lax.fori_loop(0, CH, _mul, 0)
            pltpu.sync_copy(rowsv, acc.at[idxa.at[0]], add=True)
            return 0
        lax.fori_loop(0, EPT // CH, chunk, 0)
        plsc.subcore_barrier()
        pltpu.sync_copy(acc.at[pl.ds(s * RPT, RPT)],
                        agg_hbm.at[pl.ds(lo + s * RPT, RPT)])
        if p == 0:
            pltpu.sync_copy(deg2.at[pl.ds(s * 24, 24)],
                            deg_hbm.at[c, pl.ds(s * 24, 24)])

            @pl.when(s == 0)
            def _():
                pltpu.sync_copy(deg2.at[pl.ds(384, 8)],
                                deg_hbm.at[c, pl.ds(384, 8)])
    plsc.subcore_barrier()


def _stage3(hm, gate, src, dst, ident):
    mesh = plsc.VectorSubcoreMesh(core_axis_name="c", subcore_axis_name="s")
    f = functools.partial(
        pl.kernel, _edge_body, mesh=mesh,
        out_type=[jax.ShapeDtypeStruct((N, 2 * H), jnp.float32),
                  jax.ShapeDtypeStruct((2, DR, 128), jnp.float32)],
        scratch_types=[
            pltpu.VMEM((CH,), jnp.int32),
            pltpu.VMEM((CH,), jnp.int32),
            pltpu.VMEM((3, CH), jnp.int32),
            pltpu.VMEM((CH, H), jnp.float32),
            pltpu.VMEM((CH, 2 * H), jnp.float32),
            pltpu.VMEM_SHARED((N4 + 8, 2 * H), jnp.float32),
            pltpu.VMEM_SHARED((DR + 8, 128), jnp.float32),
            pltpu.SemaphoreType.DMA,
        ],
    )()
    return f(hm, gate, src, dst, ident)


def _stage4_body(ha_ref, hb_ref, agg_ref, deg_ref, agg_w, agg_bb, nm_g,
                 nm_b, f1_w, f1_b, f2_w, f2_b, nf_g, nf_b, key_w, key_b,
                 pq, val_w, val_b, s_ref):
    i = pl.program_id(0)
    a = agg_ref[...]
    nloc = lax.broadcasted_iota(jnp.int32, (RT, 1), 0)
    rsel = (nloc // 128 == lax.broadcasted_iota(jnp.int32, (RT, 8), 1)
            ).astype(jnp.float32)
    d8 = 0.5 * jnp.sum(deg_ref[...], axis=0)
    dflat = jnp.dot(rsel, d8, preferred_element_type=jnp.float32)
    lsel = (nloc % 128 == lax.broadcasted_iota(jnp.int32, (RT, 128), 1))
    deg = jnp.sum(jnp.where(lsel, dflat, 0.0), axis=1, keepdims=True)
    rdeg = 1.0 / jnp.clip(deg, 1.0, None)
    n = i * RT + nloc
    y = (n // WD).astype(jnp.float32) * (1.0 / (HT - 1))
    x = (n % WD).astype(jnp.float32) * (1.0 / (WD - 1))
    lane = lax.broadcasted_iota(jnp.int32, (RT, H), 1)
    aux = (jnp.where(lane == 0, y, 0.0) + jnp.where(lane == 1, x, 0.0)
           + jnp.where(lane == 2, 1.0, 0.0))

    @pl.when(i == 0)
    def _():
        s_ref[...] = jnp.zeros_like(s_ref)

    for b, h_ref in ((0, ha_ref), (1, hb_ref)):
        aggn = a[:, b * H:(b + 1) * H] * rdeg
        t = _gelu(_dot_t(aggn, agg_w[...]) + agg_bb[...])
        h2 = _ln(h_ref[...] + t, nm_g[...], nm_b[...])
        ffn = (_dot_t(_gelu(_dot_t(h2, f1_w[...]) + f1_b[...]), f2_w[...])
               + f2_b[...])
        h3 = _ln(h2 + ffn, nf_g[...], nf_b[...])
        keys = _dot_t(h3, key_w[...]) + key_b[...]
        logits = _dot_t(keys, pq[...]) * 0.125
        z = logits - jnp.max(logits, axis=-1, keepdims=True)
        ez = jnp.exp(z)
        m = ez / jnp.sum(ez, axis=-1, keepdims=True)
        vals = _dot_t(h3, val_w[...]) + val_b[...]
        va = jnp.concatenate([vals, aux], axis=1)
        part = lax.dot_general(m, va, (((0,), (0,)), ((), ())),
                               preferred_element_type=jnp.float32)
        s_ref[b, :, :] += part


def _stage4(ha, hb, agg, deg, p):
    full = lambda s: pl.BlockSpec(s, lambda i: (0,) * len(s))
    r1 = lambda name: p[name].reshape(1, -1)
    return pl.pallas_call(
        _stage4_body,
        grid=(NT,),
        in_specs=[
            pl.BlockSpec((RT, H), lambda i: (i, 0)),
            pl.BlockSpec((RT, H), lambda i: (i, 0)),
            pl.BlockSpec((RT, 2 * H), lambda i: (i, 0)),
            pl.BlockSpec((2, 8, 128), lambda i: (0, i, 0)),
            full((H, H)), full((1, H)), full((1, H)), full((1, H)),
            full((2 * H, H)), full((1, 2 * H)), full((H, 2 * H)),
            full((1, H)), full((1, H)), full((1, H)),
            full((H, H)), full((1, H)), full((K, H)),
            full((H, H)), full((1, H)),
        ],
        out_specs=pl.BlockSpec((B, K, 2 * H), lambda i: (0, 0, 0)),
        out_shape=jax.ShapeDtypeStruct((B, K, 2 * H), jnp.float32),
    )(ha, hb, agg, deg, p['agg_w'], r1('agg_b'), r1('nm_g'), r1('nm_b'),
      p['f1_w'], r1('f1_b'), p['f2_w'], r1('f2_b'), r1('nf_g'), r1('nf_b'),
      p['key_w'], r1('key_b'), p['pq'], p['val_w'], r1('val_b'))


def _stage5_body(s_ref, pos1_wt, pos1_b, pos2_w, pos2_b,
                 qw, qb, kw, kb, vw, vb, ao_w, ao_b, na_g, na_b,
                 pf1_w, pf1_b, pf2_w, pf2_b, np_g, np_b,
                 c1_w, c1_b, c2_w, c2_b, out_ref):
    lane = lax.broadcasted_iota(jnp.int32, (K, 2 * H), 1)
    for b in range(B):
        sb = s_ref[b]
        den = jnp.sum(jnp.where(lane == H + 2, sb, 0.0), axis=1,
                      keepdims=True) + 1e-8
        cy = jnp.sum(jnp.where(lane == H, sb, 0.0), axis=1, keepdims=True)
        cx = jnp.sum(jnp.where(lane == H + 1, sb, 0.0), axis=1, keepdims=True)
        pf = sb[:, 0:H] / den
        pe = _gelu((cy / den) * pos1_wt[0:1, :] + (cx / den) * pos1_wt[1:2, :]
                   + pos1_b[...])
        pf = pf + _dot_t(pe, pos2_w[...]) + pos2_b[...]
        q = _dot_t(pf, qw[...]) + qb[...]
        k = _dot_t(pf, kw[...]) + kb[...]
        v = _dot_t(pf, vw[...]) + vb[...]
        dh = H // HEADS
        aos = []
        for hh in range(HEADS):
            qh = q[:, hh * dh:(hh + 1) * dh]
            kh = k[:, hh * dh:(hh + 1) * dh]
            vh = v[:, hh * dh:(hh + 1) * dh]
            aw = lax.dot_general(qh, kh, (((1,), (1,)), ((), ())),
                                 preferred_element_type=jnp.float32) * 0.25
            aw = aw - jnp.max(aw, axis=-1, keepdims=True)
            ea = jnp.exp(aw)
            aw = ea / jnp.sum(ea, axis=-1, keepdims=True)
            aos.append(jnp.dot(aw, vh, preferred_element_type=jnp.float32))
        ao = jnp.concatenate(aos, axis=1)
        ao = _dot_t(ao, ao_w[...]) + ao_b[...]
        hp = _ln(pf + ao, na_g[...], na_b[...])
        pffn = _dot_t(_gelu(_dot_t(hp, pf1_w[...]) + pf1_b[...]),
                      pf2_w[...]) + pf2_b[...]
        hp = _ln(hp + pffn, np_g[...], np_b[...])
        img = jnp.mean(hp, axis=0, keepdims=True)
        ob = _dot_t(_gelu(_dot_t(img, c1_w[...]) + c1_b[...]),
                    c2_w[...]) + c2_b[...]
        out_ref[b:b + 1, :] = ob


def _stage5(s, p):
    c2p = jnp.zeros((2 * H, 2 * H), jnp.float32).at[:C, :].set(p['c2_w'])
    c2bp = jnp.zeros((1, 2 * H), jnp.float32).at[0, :C].set(p['c2_b'])
    r1 = lambda a: a.reshape(1, -1)
    args = [s, p['pos1_w'].T, r1(p['pos1_b']), p['pos2_w'], r1(p['pos2_b']),
            p['qkv_w'][0:H], r1(p['qkv_b'][0:H]),
            p['qkv_w'][H:2 * H], r1(p['qkv_b'][H:2 * H]),
            p['qkv_w'][2 * H:], r1(p['qkv_b'][2 * H:]),
            p['ao_w'], r1(p['ao_b']), r1(p['na_g']), r1(p['na_b']),
            p['pf1_w'], r1(p['pf1_b']), p['pf2_w'], r1(p['pf2_b']),
            r1(p['np_g']), r1(p['np_b']),
            p['c1_w'], r1(p['c1_b']), c2p, c2bp]
    out = pl.pallas_call(
        _stage5_body,
        in_specs=[pl.BlockSpec(a.shape, (lambda nd: lambda: (0,) * nd)(a.ndim))
                  for a in args],
        out_specs=pl.BlockSpec((B, 2 * H), lambda: (0, 0)),
        out_shape=jax.ShapeDtypeStruct((B, 2 * H), jnp.float32),
    )(*args)
    return out[:, :C]


def kernel(x, edge_index, edge_attr, params):
    p = params
    x2 = x.reshape(B * N, D_IN)
    src = edge_index[0].astype(jnp.int32)
    dst = edge_index[1].astype(jnp.int32)
    ha, hb, hm = _stage1(x2, p)
    gate = _stage2(edge_attr, p)
    ident = jnp.eye(128, dtype=jnp.float32)
    agg, deg = _stage3(hm, gate, src, dst, ident)
    s = _stage4(ha, hb, agg, deg, p)
    return _stage5(s, p)


# speedup vs baseline: 14.0781x; 1.1128x over previous
import functools

import jax
import jax.numpy as jnp
from jax import lax
from jax.experimental import pallas as pl
from jax.experimental.pallas import tpu as pltpu, tpu_sc as plsc

B, N, D_IN, E, ED, H = 2, 50176, 7, 401408, 5, 64
K, HEADS, C, HT, WD = 16, 4, 7, 224, 224

RT = 1024
NT = N // RT
ET = 2048

N4 = N // 4
CH = 64
NSUB = 16
EPT = E // NSUB
RPT = N4 // NSUB
NPT = N // 32


def _ln(x, g, b):
    m = jnp.mean(x, axis=-1, keepdims=True)
    v = jnp.mean((x - m) ** 2, axis=-1, keepdims=True)
    return (x - m) * jax.lax.rsqrt(v + 1e-5) * g + b


def _gelu(x):
    return x * 0.5 * (1.0 + lax.erf(x * 0.7071067811865476))


def _dot_t(x, w):
    return lax.dot_general(x, w, (((1,), (1,)), ((), ())),
                           preferred_element_type=jnp.float32)


def _stage1_body(xa_ref, xb_ref, in_w, in_b, in_g, in_lb, msg_w, msg_b,
                 ha_ref, hb_ref, hm_ref):
    ha = _gelu(_ln(_dot_t(xa_ref[...], in_w[...]) + in_b[...],
                   in_g[...], in_lb[...]))
    hb = _gelu(_ln(_dot_t(xb_ref[...], in_w[...]) + in_b[...],
                   in_g[...], in_lb[...]))
    ha_ref[...] = ha
    hb_ref[...] = hb
    hm_ref[...] = jnp.concatenate(
        [_gelu(_dot_t(ha, msg_w[...]) + msg_b[...]),
         _gelu(_dot_t(hb, msg_w[...]) + msg_b[...])], axis=1)


def _stage1(x2, p):
    full = lambda s: pl.BlockSpec(s, lambda i: (0,) * len(s))
    return pl.pallas_call(
        _stage1_body,
        grid=(NT,),
        in_specs=[
            pl.BlockSpec((RT, D_IN), lambda i: (i, 0)),
            pl.BlockSpec((RT, D_IN), lambda i: (NT + i, 0)),
            full((H, D_IN)), full((1, H)), full((1, H)), full((1, H)),
            full((H, H)), full((1, H)),
        ],
        out_specs=[
            pl.BlockSpec((RT, H), lambda i: (i, 0)),
            pl.BlockSpec((RT, H), lambda i: (i, 0)),
            pl.BlockSpec((RT, 2 * H), lambda i: (i, 0)),
        ],
        out_shape=[
            jax.ShapeDtypeStruct((N, H), jnp.float32),
            jax.ShapeDtypeStruct((N, H), jnp.float32),
            jax.ShapeDtypeStruct((N, 2 * H), jnp.float32),
        ],
    )(x2, x2, p['in_w'], p['in_b'].reshape(1, H), p['in_g'].reshape(1, H),
      p['in_lb'].reshape(1, H), p['msg_w'], p['msg_b'].reshape(1, H))


def _stage2_body(ea_ref, w1, b1, w2, b2, gate_ref):
    t = _gelu(_dot_t(ea_ref[...], w1[...]) + b1[...])
    gate_ref[...] = jax.nn.sigmoid(_dot_t(t, w2[...]) + b2[...])


def _stage2(edge_attr, p):
    full = lambda s: pl.BlockSpec(s, lambda i: (0,) * len(s))
    return pl.pallas_call(
        _stage2_body,
        grid=(E // ET,),
        in_specs=[
            pl.BlockSpec((ET, ED), lambda i: (i, 0)),
            full((H, ED)), full((1, H)), full((H, H)), full((1, H)),
        ],
        out_specs=pl.BlockSpec((ET, H), lambda i: (i, 0)),
        out_shape=jax.ShapeDtypeStruct((E, H), jnp.float32),
    )(edge_attr, p['eg1_w'], p['eg1_b'].reshape(1, H), p['eg2_w'],
      p['eg2_b'].reshape(1, H))


DR = N // 128


def _edge_body(hm_hbm, gate_hbm, src_hbm, dst_hbm, id_hbm, agg_hbm, deg_hbm,
               srcv, dstv, idxa, gatev, rowsv, acc, deg2, sem):
    c = lax.axis_index("c")
    s = lax.axis_index("s")

    for p in range(2):
        lo = (c * 2 + p) * N4
        plsc.subcore_barrier()

        def _z(i, _):
            rowsv[i // 8, pl.ds((i % 8) * 16, 16)] = jnp.zeros((16,),
                                                              jnp.float32)
            return 0
        lax.fori_loop(0, CH * 8, _z, 0)
        for z in range(RPT // CH):
            pltpu.sync_copy(rowsv, acc.at[pl.ds(s * RPT + z * CH, CH)])
        if RPT % CH:
            pltpu.sync_copy(rowsv.at[pl.ds(0, RPT % CH)],
                            acc.at[pl.ds(s * RPT + (RPT // CH) * CH,
                                         RPT % CH)])

        @pl.when(s == 0)
        def _():
            pltpu.sync_copy(rowsv.at[pl.ds(0, 8)], acc.at[pl.ds(N4, 8)])
        if p == 0:
            pltpu.sync_copy(rowsv.at[pl.ds(0, 24)],
                            deg2.at[pl.ds(s * 24, 24)])

            @pl.when(s == 0)
            def _():
                pltpu.sync_copy(rowsv.at[pl.ds(0, 8)],
                                deg2.at[pl.ds(384, 8)])
        plsc.subcore_barrier()

        def chunk(ci, _):
            base = s * EPT + ci * CH
            pltpu.sync_copy(src_hbm.at[pl.ds(base, CH)], srcv)
            hmcp = pltpu.async_copy(hm_hbm.at[srcv], rowsv, sem)
            pltpu.sync_copy(dst_hbm.at[pl.ds(base, CH)], dstv)
            pltpu.sync_copy(gate_hbm.at[pl.ds(base, CH)], gatev)

            def _i1(j, _):
                d = dstv[pl.ds(j * 16, 16)]
                lcl = d - lo
                m = (lcl >= 0) & (lcl < N4)
                idxa[0, pl.ds(j * 16, 16)] = jnp.where(m, lcl, N4)
                return 0
            lax.fori_loop(0, CH // 16, _i1, 0)
            if p == 0:
                def _i2(j, _):
                    d = dstv[pl.ds(j * 16, 16)]
                    idxa[1, pl.ds(j * 16, 16)] = d >> 7
                    return 0
                lax.fori_loop(0, CH // 16, _i2, 0)

                def _i3(j, _):
                    d = dstv[pl.ds(j * 16, 16)]
                    idxa[2, pl.ds(j * 16, 16)] = d & 127
                    return 0
                lax.fori_loop(0, CH // 16, _i3, 0)
            hmcp.wait()

            def _mul(e, _):
                g = [gatev[e, pl.ds(kk * 16, 16)] for kk in range(4)]
                for kk in range(8):
                    rowsv[e, pl.ds(kk * 16, 16)] = (
                        rowsv[e, pl.ds(kk * 16, 16)] * g[kk % 4])
                return 0
            lax.fori_loop(0, CH, _mul, 0)
            pltpu.sync_copy(rowsv, acc.at[idxa.at[0]], add=True)
            if p == 0:
                pltpu.async_copy(id_hbm.at[idxa.at[2]], rowsv, sem).wait()
                pltpu.sync_copy(rowsv, deg2.at[idxa.at[1]], add=True)
            return 0
        lax.fori_loop(0, EPT // CH, chunk, 0)
        plsc.subcore_barrier()
        pltpu.sync_copy(acc.at[pl.ds(s * RPT, RPT)],
                        agg_hbm.at[pl.ds(lo + s * RPT, RPT)])
        if p == 0:
            pltpu.sync_copy(deg2.at[pl.ds(s * 24, 24)],
                            deg_hbm.at[c, pl.ds(s * 24, 24)])

            @pl.when(s == 0)
            def _():
                pltpu.sync_copy(deg2.at[pl.ds(384, 8)],
                                deg_hbm.at[c, pl.ds(384, 8)])
    plsc.subcore_barrier()


def _stage3(hm, gate, src, dst, ident):
    mesh = plsc.VectorSubcoreMesh(core_axis_name="c", subcore_axis_name="s")
    f = functools.partial(
        pl.kernel, _edge_body, mesh=mesh,
        out_type=[jax.ShapeDtypeStruct((N, 2 * H), jnp.float32),
                  jax.ShapeDtypeStruct((2, DR, 128), jnp.float32)],
        scratch_types=[
            pltpu.VMEM((CH,), jnp.int32),
            pltpu.VMEM((CH,), jnp.int32),
            pltpu.VMEM((3, CH), jnp.int32),
            pltpu.VMEM((CH, H), jnp.float32),
            pltpu.VMEM((CH, 2 * H), jnp.float32),
            pltpu.VMEM_SHARED((N4 + 8, 2 * H), jnp.float32),
            pltpu.VMEM_SHARED((DR + 8, 128), jnp.float32),
            pltpu.SemaphoreType.DMA,
        ],
    )()
    return f(hm, gate, src, dst, ident)


def _stage4_body(ha_ref, hb_ref, agg_ref, deg_ref, agg_w, agg_bb, nm_g,
                 nm_b, f1_w, f1_b, f2_w, f2_b, nf_g, nf_b, key_w, key_b,
                 pq, val_w, val_b, s_ref):
    i = pl.program_id(0)
    a = agg_ref[...]
    nloc = lax.broadcasted_iota(jnp.int32, (RT, 1), 0)
    rsel = (nloc // 128 == lax.broadcasted_iota(jnp.int32, (RT, 8), 1)
            ).astype(jnp.float32)
    d8 = 0.5 * jnp.sum(deg_ref[...], axis=0)
    dflat = jnp.dot(rsel, d8, preferred_element_type=jnp.float32)
    lsel = (nloc % 128 == lax.broadcasted_iota(jnp.int32, (RT, 128), 1))
    deg = jnp.sum(jnp.where(lsel, dflat, 0.0), axis=1, keepdims=True)
    rdeg = 1.0 / jnp.clip(deg, 1.0, None)
    n = i * RT + nloc
    y = (n // WD).astype(jnp.float32) * (1.0 / (HT - 1))
    x = (n % WD).astype(jnp.float32) * (1.0 / (WD - 1))
    lane = lax.broadcasted_iota(jnp.int32, (RT, H), 1)
    aux = (jnp.where(lane == 0, y, 0.0) + jnp.where(lane == 1, x, 0.0)
           + jnp.where(lane == 2, 1.0, 0.0))

    @pl.when(i == 0)
    def _():
        s_ref[...] = jnp.zeros_like(s_ref)

    for b, h_ref in ((0, ha_ref), (1, hb_ref)):
        aggn = a[:, b * H:(b + 1) * H] * rdeg
        t = _gelu(_dot_t(aggn, agg_w[...]) + agg_bb[...])
        h2 = _ln(h_ref[...] + t, nm_g[...], nm_b[...])
        ffn = (_dot_t(_gelu(_dot_t(h2, f1_w[...]) + f1_b[...]), f2_w[...])
               + f2_b[...])
        h3 = _ln(h2 + ffn, nf_g[...], nf_b[...])
        keys = _dot_t(h3, key_w[...]) + key_b[...]
        logits = _dot_t(keys, pq[...]) * 0.125
        z = logits - jnp.max(logits, axis=-1, keepdims=True)
        ez = jnp.exp(z)
        m = ez / jnp.sum(ez, axis=-1, keepdims=True)
        vals = _dot_t(h3, val_w[...]) + val_b[...]
        va = jnp.concatenate([vals, aux], axis=1)
        part = lax.dot_general(m, va, (((0,), (0,)), ((), ())),
                               preferred_element_type=jnp.float32)
        s_ref[b, :, :] += part


def _stage4(ha, hb, agg, deg, p):
    full = lambda s: pl.BlockSpec(s, lambda i: (0,) * len(s))
    r1 = lambda name: p[name].reshape(1, -1)
    return pl.pallas_call(
        _stage4_body,
        grid=(NT,),
        in_specs=[
            pl.BlockSpec((RT, H), lambda i: (i, 0)),
            pl.BlockSpec((RT, H), lambda i: (i, 0)),
            pl.BlockSpec((RT, 2 * H), lambda i: (i, 0)),
            pl.BlockSpec((2, 8, 128), lambda i: (0, i, 0)),
            full((H, H)), full((1, H)), full((1, H)), full((1, H)),
            full((2 * H, H)), full((1, 2 * H)), full((H, 2 * H)),
            full((1, H)), full((1, H)), full((1, H)),
            full((H, H)), full((1, H)), full((K, H)),
            full((H, H)), full((1, H)),
        ],
        out_specs=pl.BlockSpec((B, K, 2 * H), lambda i: (0, 0, 0)),
        out_shape=jax.ShapeDtypeStruct((B, K, 2 * H), jnp.float32),
    )(ha, hb, agg, deg, p['agg_w'], r1('agg_b'), r1('nm_g'), r1('nm_b'),
      p['f1_w'], r1('f1_b'), p['f2_w'], r1('f2_b'), r1('nf_g'), r1('nf_b'),
      p['key_w'], r1('key_b'), p['pq'], p['val_w'], r1('val_b'))


def _stage5_body(s_ref, pos1_wt, pos1_b, pos2_w, pos2_b,
                 qw, qb, kw, kb, vw, vb, ao_w, ao_b, na_g, na_b,
                 pf1_w, pf1_b, pf2_w, pf2_b, np_g, np_b,
                 c1_w, c1_b, c2_w, c2_b, out_ref):
    lane = lax.broadcasted_iota(jnp.int32, (K, 2 * H), 1)
    for b in range(B):
        sb = s_ref[b]
        den = jnp.sum(jnp.where(lane == H + 2, sb, 0.0), axis=1,
                      keepdims=True) + 1e-8
        cy = jnp.sum(jnp.where(lane == H, sb, 0.0), axis=1, keepdims=True)
        cx = jnp.sum(jnp.where(lane == H + 1, sb, 0.0), axis=1, keepdims=True)
        pf = sb[:, 0:H] / den
        pe = _gelu((cy / den) * pos1_wt[0:1, :] + (cx / den) * pos1_wt[1:2, :]
                   + pos1_b[...])
        pf = pf + _dot_t(pe, pos2_w[...]) + pos2_b[...]
        q = _dot_t(pf, qw[...]) + qb[...]
        k = _dot_t(pf, kw[...]) + kb[...]
        v = _dot_t(pf, vw[...]) + vb[...]
        dh = H // HEADS
        aos = []
        for hh in range(HEADS):
            qh = q[:, hh * dh:(hh + 1) * dh]
            kh = k[:, hh * dh:(hh + 1) * dh]
            vh = v[:, hh * dh:(hh + 1) * dh]
            aw = lax.dot_general(qh, kh, (((1,), (1,)), ((), ())),
                                 preferred_element_type=jnp.float32) * 0.25
            aw = aw - jnp.max(aw, axis=-1, keepdims=True)
            ea = jnp.exp(aw)
            aw = ea / jnp.sum(ea, axis=-1, keepdims=True)
            aos.append(jnp.dot(aw, vh, preferred_element_type=jnp.float32))
        ao = jnp.concatenate(aos, axis=1)
        ao = _dot_t(ao, ao_w[...]) + ao_b[...]
        hp = _ln(pf + ao, na_g[...], na_b[...])
        pffn = _dot_t(_gelu(_dot_t(hp, pf1_w[...]) + pf1_b[...]),
                      pf2_w[...]) + pf2_b[...]
        hp = _ln(hp + pffn, np_g[...], np_b[...])
        img = jnp.mean(hp, axis=0, keepdims=True)
        ob = _dot_t(_gelu(_dot_t(img, c1_w[...]) + c1_b[...]),
                    c2_w[...]) + c2_b[...]
        out_ref[b:b + 1, :] = ob


def _stage5(s, p):
    c2p = jnp.zeros((2 * H, 2 * H), jnp.float32).at[:C, :].set(p['c2_w'])
    c2bp = jnp.zeros((1, 2 * H), jnp.float32).at[0, :C].set(p['c2_b'])
    r1 = lambda a: a.reshape(1, -1)
    args = [s, p['pos1_w'].T, r1(p['pos1_b']), p['pos2_w'], r1(p['pos2_b']),
            p['qkv_w'][0:H], r1(p['qkv_b'][0:H]),
            p['qkv_w'][H:2 * H], r1(p['qkv_b'][H:2 * H]),
            p['qkv_w'][2 * H:], r1(p['qkv_b'][2 * H:]),
            p['ao_w'], r1(p['ao_b']), r1(p['na_g']), r1(p['na_b']),
            p['pf1_w'], r1(p['pf1_b']), p['pf2_w'], r1(p['pf2_b']),
            r1(p['np_g']), r1(p['np_b']),
            p['c1_w'], r1(p['c1_b']), c2p, c2bp]
    out = pl.pallas_call(
        _stage5_body,
        in_specs=[pl.BlockSpec(a.shape, (lambda nd: lambda: (0,) * nd)(a.ndim))
                  for a in args],
        out_specs=pl.BlockSpec((B, 2 * H), lambda: (0, 0)),
        out_shape=jax.ShapeDtypeStruct((B, 2 * H), jnp.float32),
    )(*args)
    return out[:, :C]


def kernel(x, edge_index, edge_attr, params):
    p = params
    x2 = x.reshape(B * N, D_IN)
    src = edge_index[0].astype(jnp.int32)
    dst = edge_index[1].astype(jnp.int32)
    ha, hb, hm = _stage1(x2, p)
    gate = _stage2(edge_attr, p)
    ident = jnp.eye(128, dtype=jnp.float32)
    agg, deg = _stage3(hm, gate, src, dst, ident)
    s = _stage4(ha, hb, agg, deg, p)
    return _stage5(s, p)
